# Initial kernel scaffold; baseline (speedup 1.0000x reference)
import jax
import jax.numpy as jnp
from jax.experimental import pallas as pl


def kernel(x, edge_indices, edge_weights, W1, b1, W2, b2, attention):
    raise NotImplementedError("write your pallas kernel here")


# speedup vs baseline: 55.0640x; 55.0640x over previous
import functools

import jax
import jax.numpy as jnp
from jax import lax
from jax.experimental import pallas as pl
from jax.experimental.pallas import tpu as pltpu
from jax.experimental.pallas import tpu_sc as plsc

N = 10000
E = 320000
V = 3
D_IN = 256
D_OUT = 3
H = 2 * D_OUT
W = 8

CHUNK = 1024
E_PAD = 327680
PER_TILE = E_PAD // 32
N_CHUNKS = PER_TILE // CHUNK

ROWS_PER = 624
TAIL_START = 16 * ROWS_PER
TAIL = N - TAIL_START
RBUF = ROWS_PER + TAIL

_SC_PARAMS = pltpu.CompilerParams(
    use_tc_tiling_on_sc=False, needs_layout_passes=False)


def _sc_mesh():
    return plsc.VectorSubcoreMesh(core_axis_name="c", subcore_axis_name="s",
                                  num_cores=2, num_subcores=16)


_LANE = None


def _rsqrt16(x):
    i = plsc.bitcast(x, jnp.int32)
    y = plsc.bitcast(jnp.int32(0x5F3759DF) - (i >> 1), jnp.float32)
    for _ in range(3):
        y = y * (1.5 - 0.5 * x * y * y)
    return y


def _ld2(ref, j, eoff, feat):
    return plsc.load_gather(ref, [2 * j + eoff, feat])


def _st2(ref, j, x, eoff, feat):
    plsc.store_scatter(ref, [2 * j + eoff, feat], x)


@functools.partial(
    pl.kernel,
    out_type=jax.ShapeDtypeStruct((2, V, N, W), jnp.float32),
    mesh=_sc_mesh(),
    compiler_params=_SC_PARAMS,
    scratch_types=[
        pltpu.VMEM_SHARED((N, W), jnp.float32),
        pltpu.VMEM_SHARED((N, W), jnp.float32),
        pltpu.VMEM_SHARED((N, W), jnp.float32),
        pltpu.VMEM((CHUNK,), jnp.int32),
        pltpu.VMEM((CHUNK,), jnp.float32),
        pltpu.VMEM((CHUNK, W), jnp.float32),
    ],
)
def _deg_kernel(col_hbm, ew_hbm, zero8_hbm, out_hbm, acc0, acc1, acc2,
                col_buf, ew_buf, dv):
    c = lax.axis_index("c")
    s = lax.axis_index("s")
    accs = (acc0, acc1, acc2)

    for acc in accs:
        pltpu.sync_copy(zero8_hbm.at[pl.ds(s * ROWS_PER, ROWS_PER), :],
                        acc.at[pl.ds(s * ROWS_PER, ROWS_PER), :])

        @pl.when(s == 15)
        def _():
            pltpu.sync_copy(zero8_hbm.at[pl.ds(TAIL_START, TAIL), :],
                            acc.at[pl.ds(TAIL_START, TAIL), :])

    plsc.subcore_barrier()

    lane = lax.iota(jnp.int32, 16)
    eoff = lane >> 3
    feat = lane & 7

    for v in range(V):
        acc = accs[v]

        def chunk_body(k, _, acc=acc, v=v):
            base = v * E_PAD + c * (E_PAD // 2) + s * PER_TILE + k * CHUNK
            pltpu.sync_copy(col_hbm.at[pl.ds(base, CHUNK)], col_buf)
            pltpu.sync_copy(ew_hbm.at[pl.ds(base, CHUNK)], ew_buf)

            @plsc.parallel_loop(0, CHUNK // 2, unroll=8)
            def _(i):
                w = plsc.load_gather(ew_buf, [2 * i + eoff])
                _st2(dv, i, w, eoff, feat)
            pltpu.sync_copy(dv, acc.at[col_buf], add=True)
            return 0

        lax.fori_loop(0, N_CHUNKS, chunk_body, 0)

    plsc.subcore_barrier()

    for v in range(V):
        acc = accs[v]
        pltpu.sync_copy(acc.at[pl.ds(s * ROWS_PER, ROWS_PER), :],
                        out_hbm.at[c, v, pl.ds(s * ROWS_PER, ROWS_PER), :])

        @pl.when(s == 15)
        def _(v=v):
            pltpu.sync_copy(acc.at[pl.ds(TAIL_START, TAIL), :],
                            out_hbm.at[c, v, pl.ds(TAIL_START, TAIL), :])


def _ranges(s):
    return [(s * ROWS_PER, ROWS_PER, 0), (TAIL_START, TAIL, ROWS_PER)]


def _edge_sweep(c, s, row_hbm, col_hbm, ew_hbm, zps, accs,
                row_buf, col_buf, ew_buf, gv, eoff, feat):
    for v in range(V):
        zp = zps[v]
        acc = accs[v]

        def chunk_body(k, _, zp=zp, acc=acc, v=v):
            base = v * E_PAD + c * (E_PAD // 2) + s * PER_TILE + k * CHUNK
            pltpu.sync_copy(row_hbm.at[pl.ds(base, CHUNK)], row_buf)
            pltpu.sync_copy(col_hbm.at[pl.ds(base, CHUNK)], col_buf)
            pltpu.sync_copy(ew_hbm.at[pl.ds(base, CHUNK)], ew_buf)
            pltpu.sync_copy(zp.at[row_buf], gv)

            @plsc.parallel_loop(0, CHUNK // 2, unroll=8)
            def _(i):
                w = plsc.load_gather(ew_buf, [2 * i + eoff])
                g = _ld2(gv, i, eoff, feat)
                _st2(gv, i, g * w, eoff, feat)
            pltpu.sync_copy(gv, acc.at[col_buf], add=True)
            return 0

        lax.fori_loop(0, N_CHUNKS, chunk_body, 0)


_PROP_SCRATCH = [
    pltpu.VMEM_SHARED((N, W), jnp.float32),
    pltpu.VMEM_SHARED((N, W), jnp.float32),
    pltpu.VMEM_SHARED((N, W), jnp.float32),
    pltpu.VMEM_SHARED((N, W), jnp.float32),
    pltpu.VMEM_SHARED((N, W), jnp.float32),
    pltpu.VMEM_SHARED((N, W), jnp.float32),
    pltpu.VMEM((CHUNK,), jnp.int32),
    pltpu.VMEM((CHUNK,), jnp.int32),
    pltpu.VMEM((CHUNK,), jnp.float32),
    pltpu.VMEM((CHUNK, W), jnp.float32),
    pltpu.VMEM((RBUF, W), jnp.float32),
    pltpu.VMEM((RBUF, W), jnp.float32),
    pltpu.VMEM((RBUF, W), jnp.float32),
]


@functools.partial(
    pl.kernel,
    out_type=jax.ShapeDtypeStruct((2, V, N, W), jnp.float32),
    mesh=_sc_mesh(),
    compiler_params=_SC_PARAMS,
    scratch_types=_PROP_SCRATCH,
)
def _prop1_kernel(degp_hbm, xw_hbm, row_hbm, col_hbm, ew_hbm, zero8_hbm,
                  out_hbm, zp0, zp1, zp2, acc0, acc1, acc2,
                  row_buf, col_buf, ew_buf, gv, p0b, p1b, xwb):
    c = lax.axis_index("c")
    s = lax.axis_index("s")
    zps = (zp0, zp1, zp2)
    accs = (acc0, acc1, acc2)
    lane = lax.iota(jnp.int32, 16)
    eoff = lane >> 3
    feat = lane & 7

    for v in range(V):
        pltpu.sync_copy(zero8_hbm.at[pl.ds(s * ROWS_PER, ROWS_PER), :],
                        accs[v].at[pl.ds(s * ROWS_PER, ROWS_PER), :])

        @pl.when(s == 15)
        def _(v=v):
            pltpu.sync_copy(zero8_hbm.at[pl.ds(TAIL_START, TAIL), :],
                            accs[v].at[pl.ds(TAIL_START, TAIL), :])

        for start, nrows, _boff in _ranges(s):
            def do_range(start=start, nrows=nrows, v=v):
                pltpu.sync_copy(degp_hbm.at[0, v, pl.ds(start, nrows), :],
                                p0b.at[pl.ds(0, nrows), :])
                pltpu.sync_copy(degp_hbm.at[1, v, pl.ds(start, nrows), :],
                                p1b.at[pl.ds(0, nrows), :])
                pltpu.sync_copy(xw_hbm.at[v, pl.ds(start, nrows), :],
                                xwb.at[pl.ds(0, nrows), :])

                @plsc.parallel_loop(0, nrows * W // 16, unroll=4)
                def _(j):
                    d = _ld2(p0b, j, eoff, feat) + _ld2(p1b, j, eoff, feat) + 1.0
                    y = _rsqrt16(d)
                    _st2(xwb, j, y * _ld2(xwb, j, eoff, feat), eoff, feat)

                pltpu.sync_copy(xwb.at[pl.ds(0, nrows), :],
                                zps[v].at[pl.ds(start, nrows), :])

            if _boff == 0:
                do_range()
            else:
                pl.when(s == 15)(do_range)

    plsc.subcore_barrier()
    _edge_sweep(c, s, row_hbm, col_hbm, ew_hbm, zps, accs,
                row_buf, col_buf, ew_buf, gv, eoff, feat)
    plsc.subcore_barrier()

    for v in range(V):
        acc = accs[v]
        pltpu.sync_copy(acc.at[pl.ds(s * ROWS_PER, ROWS_PER), :],
                        out_hbm.at[c, v, pl.ds(s * ROWS_PER, ROWS_PER), :])

        @pl.when(s == 15)
        def _(v=v):
            pltpu.sync_copy(acc.at[pl.ds(TAIL_START, TAIL), :],
                            out_hbm.at[c, v, pl.ds(TAIL_START, TAIL), :])


@functools.partial(
    pl.kernel,
    out_type=jax.ShapeDtypeStruct((2, V, N, W), jnp.float32),
    mesh=_sc_mesh(),
    compiler_params=_SC_PARAMS,
    scratch_types=_PROP_SCRATCH + [
        pltpu.VMEM((RBUF, W), jnp.float32),
        pltpu.VMEM((RBUF, W), jnp.float32),
        pltpu.VMEM((V * RBUF * W,), jnp.float32),
        pltpu.VMEM((V * RBUF * W,), jnp.float32),
        pltpu.VMEM((16,), jnp.float32),
    ],
)
def _prop2_kernel(degp_hbm, xw_hbm, t1p_hbm, b1x_hbm, row_hbm, col_hbm,
                  ew_hbm, zero8_hbm, out_hbm,
                  zp0, zp1, zp2, acc0, acc1, acc2,
                  row_buf, col_buf, ew_buf, gv, p0b, p1b, xwb,
                  t0b, t1b, dis_t, hh, b1b):
    c = lax.axis_index("c")
    s = lax.axis_index("s")
    zps = (zp0, zp1, zp2)
    accs = (acc0, acc1, acc2)
    lane = lax.iota(jnp.int32, 16)
    eoff = lane >> 3
    feat = lane & 7

    for v in range(V):
        pltpu.sync_copy(zero8_hbm.at[pl.ds(s * ROWS_PER, ROWS_PER), :],
                        accs[v].at[pl.ds(s * ROWS_PER, ROWS_PER), :])

        @pl.when(s == 15)
        def _(v=v):
            pltpu.sync_copy(zero8_hbm.at[pl.ds(TAIL_START, TAIL), :],
                            accs[v].at[pl.ds(TAIL_START, TAIL), :])

        pltpu.sync_copy(b1x_hbm.at[v], b1b)

        for start, nrows, boff in _ranges(s):
            def do_range(start=start, nrows=nrows, boff=boff, v=v):
                pltpu.sync_copy(degp_hbm.at[0, v, pl.ds(start, nrows), :],
                                p0b.at[pl.ds(0, nrows), :])
                pltpu.sync_copy(degp_hbm.at[1, v, pl.ds(start, nrows), :],
                                p1b.at[pl.ds(0, nrows), :])
                pltpu.sync_copy(xw_hbm.at[v, pl.ds(start, nrows), :],
                                xwb.at[pl.ds(0, nrows), :])
                pltpu.sync_copy(t1p_hbm.at[0, v, pl.ds(start, nrows), :],
                                t0b.at[pl.ds(0, nrows), :])
                pltpu.sync_copy(t1p_hbm.at[1, v, pl.ds(start, nrows), :],
                                t1b.at[pl.ds(0, nrows), :])
                b1v = b1b[...]
                fbase = (v * RBUF + boff) * W

                @plsc.parallel_loop(0, nrows * W // 16, unroll=4)
                def _(j):
                    d = _ld2(p0b, j, eoff, feat) + _ld2(p1b, j, eoff, feat) + 1.0
                    y = _rsqrt16(d)
                    y2 = y * y
                    pre = (y * (_ld2(t0b, j, eoff, feat) + _ld2(t1b, j, eoff, feat))
                           + y2 * _ld2(xwb, j, eoff, feat) + b1v)
                    h = jnp.maximum(pre, 0.0)
                    _st2(xwb, j, y * h, eoff, feat)
                    dis_t[pl.ds(fbase + 16 * j, 16)] = y
                    hh[pl.ds(fbase + 16 * j, 16)] = 0.5 * y2 * h

                pltpu.sync_copy(xwb.at[pl.ds(0, nrows), :],
                                zps[v].at[pl.ds(start, nrows), :])

            if boff == 0:
                do_range()
            else:
                pl.when(s == 15)(do_range)

    plsc.subcore_barrier()
    _edge_sweep(c, s, row_hbm, col_hbm, ew_hbm, zps, accs,
                row_buf, col_buf, ew_buf, gv, eoff, feat)
    plsc.subcore_barrier()

    for v in range(V):
        acc = accs[v]
        for start, nrows, boff in _ranges(s):
            def do_range(start=start, nrows=nrows, boff=boff, v=v, acc=acc):
                pltpu.sync_copy(acc.at[pl.ds(start, nrows), :],
                                p0b.at[pl.ds(0, nrows), :])
                fbase = (v * RBUF + boff) * W

                @plsc.parallel_loop(0, nrows * W // 16, unroll=4)
                def _(j):
                    y = dis_t[pl.ds(fbase + 16 * j, 16)]
                    z = hh[pl.ds(fbase + 16 * j, 16)]
                    _st2(p0b, j, y * _ld2(p0b, j, eoff, feat) + z, eoff, feat)

                pltpu.sync_copy(p0b.at[pl.ds(0, nrows), :],
                                out_hbm.at[c, v, pl.ds(start, nrows), :])

            if boff == 0:
                do_range()
            else:
                pl.when(s == 15)(do_range)


BN = 2000
E_TAIL = E_PAD - E


def _repack_body(ei_ref, ew_ref, row_ref, col_ref, ewo_ref):
    for v in range(V):
        b = v * E_PAD
        row_ref[pl.ds(b, E)] = ei_ref[v, 0, :]
        row_ref[pl.ds(b + E, E_TAIL)] = jnp.zeros((E_TAIL,), jnp.int32)
        col_ref[pl.ds(b, E)] = ei_ref[v, 1, :]
        col_ref[pl.ds(b + E, E_TAIL)] = jnp.zeros((E_TAIL,), jnp.int32)
        ewo_ref[pl.ds(b, E)] = ew_ref[v, :]
        ewo_ref[pl.ds(b + E, E_TAIL)] = jnp.zeros((E_TAIL,), jnp.float32)


def _tc_repack(ei, ew):
    return pl.pallas_call(
        _repack_body,
        out_shape=[
            jax.ShapeDtypeStruct((V * E_PAD,), jnp.int32),
            jax.ShapeDtypeStruct((V * E_PAD,), jnp.int32),
            jax.ShapeDtypeStruct((V * E_PAD,), jnp.float32),
        ],
    )(ei, ew)


def _xw_body(x_ref, w_ref, o_ref):
    o_ref[...] = jnp.dot(x_ref[...], w_ref[0],
                         preferred_element_type=jnp.float32)[None]


def _tc_xw(x, w1p):
    return pl.pallas_call(
        _xw_body,
        grid=(V, N // BN),
        in_specs=[
            pl.BlockSpec((BN, D_IN), lambda v, n: (n, 0)),
            pl.BlockSpec((1, D_IN, W), lambda v, n: (v, 0, 0)),
        ],
        out_specs=pl.BlockSpec((1, BN, W), lambda v, n: (v, n, 0)),
        out_shape=jax.ShapeDtypeStruct((V, N, W), jnp.float32),
    )(x, w1p)


def _fuse_body(m_ref, w2_ref, b2_ref, o_ref):
    acc = b2_ref[0, 0] + b2_ref[1, 0] + b2_ref[2, 0]
    acc = jnp.broadcast_to(acc[None], (BN, W))
    for v in range(V):
        p2 = m_ref[0, v] + m_ref[1, v]
        acc = acc + jnp.dot(p2, w2_ref[v], preferred_element_type=jnp.float32)
    o_ref[...] = acc


def _tc_fuse(m, w2p, b2p):
    return pl.pallas_call(
        _fuse_body,
        grid=(N // BN,),
        in_specs=[
            pl.BlockSpec((2, V, BN, W), lambda n: (0, 0, n, 0)),
            pl.BlockSpec((V, W, W), lambda n: (0, 0, 0)),
            pl.BlockSpec((V, 1, W), lambda n: (0, 0, 0)),
        ],
        out_specs=pl.BlockSpec((BN, W), lambda n: (n, 0)),
        out_shape=jax.ShapeDtypeStruct((N, W), jnp.float32),
    )(m, w2p, b2p)


def kernel(x, edge_indices, edge_weights, W1, b1, W2, b2, attention):
    del attention
    f32 = jnp.float32

    row, col, ew = _tc_repack(edge_indices, edge_weights)

    zero8 = jnp.zeros((N, W), f32)
    w1p = jnp.pad(W1, ((0, 0), (0, 0), (0, W - H)))
    b1x = jnp.tile(jnp.pad(b1, ((0, 0), (0, W - H))), (1, 2))
    w2p = jnp.pad(W2, ((0, 0), (0, W - H), (0, W - D_OUT)))
    b2p = jnp.pad(b2, ((0, 0), (0, W - D_OUT)))[:, None, :]

    xw = _tc_xw(x, w1p)
    degp = _deg_kernel(col, ew, zero8)
    t1p = _prop1_kernel(degp, xw, row, col, ew, zero8)
    m = _prop2_kernel(degp, xw, t1p, b1x, row, col, ew, zero8)
    fused = _tc_fuse(m, w2p, b2p)
    return fused[:, :D_OUT]


# speedup vs baseline: 73.2764x; 1.3307x over previous
import functools

import jax
import jax.numpy as jnp
from jax import lax
from jax.experimental import pallas as pl
from jax.experimental.pallas import tpu as pltpu
from jax.experimental.pallas import tpu_sc as plsc

N = 10000
E = 320000
V = 3
D_IN = 256
D_OUT = 3
H = 2 * D_OUT
W = 8

CHUNK = 1024
E_PAD = 327680
PER_TILE = E_PAD // 32
N_CHUNKS = PER_TILE // CHUNK

ROWS_PER = 624
TAIL_START = 16 * ROWS_PER
TAIL = N - TAIL_START
RBUF = ROWS_PER + TAIL

_SC_PARAMS = pltpu.CompilerParams(
    use_tc_tiling_on_sc=False, needs_layout_passes=False)


def _sc_mesh():
    return plsc.VectorSubcoreMesh(core_axis_name="c", subcore_axis_name="s",
                                  num_cores=2, num_subcores=16)


_LANE = None


def _rsqrt16(x):
    i = plsc.bitcast(x, jnp.int32)
    y = plsc.bitcast(jnp.int32(0x5F3759DF) - (i >> 1), jnp.float32)
    for _ in range(3):
        y = y * (1.5 - 0.5 * x * y * y)
    return y


def _ld2(ref, j, eoff, feat):
    return plsc.load_gather(ref, [2 * j + eoff, feat])


def _st2(ref, j, x, eoff, feat):
    plsc.store_scatter(ref, [2 * j + eoff, feat], x)


@functools.partial(
    pl.kernel,
    out_type=jax.ShapeDtypeStruct((2, V, N, W), jnp.float32),
    mesh=_sc_mesh(),
    compiler_params=_SC_PARAMS,
    scratch_types=[
        pltpu.VMEM_SHARED((N, W), jnp.float32),
        pltpu.VMEM((CHUNK,), jnp.int32),
        pltpu.VMEM((CHUNK,), jnp.float32),
        pltpu.VMEM((CHUNK, W), jnp.float32),
    ],
)
def _deg_kernel(col_hbm, ew_hbm, zero8_hbm, out_hbm, acc,
                col_buf, ew_buf, dv):
    c = lax.axis_index("c")
    s = lax.axis_index("s")

    lane = lax.iota(jnp.int32, 16)
    eoff = lane >> 3
    feat = lane & 7

    for v in range(V):
        pltpu.sync_copy(zero8_hbm.at[pl.ds(s * ROWS_PER, ROWS_PER), :],
                        acc.at[pl.ds(s * ROWS_PER, ROWS_PER), :])

        @pl.when(s == 15)
        def _():
            pltpu.sync_copy(zero8_hbm.at[pl.ds(TAIL_START, TAIL), :],
                            acc.at[pl.ds(TAIL_START, TAIL), :])

        plsc.subcore_barrier()

        def chunk_body(k, _, v=v):
            base = v * E_PAD + c * (E_PAD // 2) + s * PER_TILE + k * CHUNK
            pltpu.sync_copy(col_hbm.at[pl.ds(base, CHUNK)], col_buf)
            pltpu.sync_copy(ew_hbm.at[pl.ds(base, CHUNK)], ew_buf)

            @plsc.parallel_loop(0, CHUNK // 2, unroll=8)
            def _(i):
                w = plsc.load_gather(ew_buf, [2 * i + eoff])
                _st2(dv, i, w, eoff, feat)
            pltpu.sync_copy(dv, acc.at[col_buf], add=True)
            return 0

        lax.fori_loop(0, N_CHUNKS, chunk_body, 0)

        plsc.subcore_barrier()

        pltpu.sync_copy(acc.at[pl.ds(s * ROWS_PER, ROWS_PER), :],
                        out_hbm.at[c, v, pl.ds(s * ROWS_PER, ROWS_PER), :])

        @pl.when(s == 15)
        def _(v=v):
            pltpu.sync_copy(acc.at[pl.ds(TAIL_START, TAIL), :],
                            out_hbm.at[c, v, pl.ds(TAIL_START, TAIL), :])

        if v + 1 < V:
            plsc.subcore_barrier()


def _ranges(s):
    return [(s * ROWS_PER, ROWS_PER, 0), (TAIL_START, TAIL, ROWS_PER)]


NSLOT = 4


def _edge_sweep(c, s, v, row_hbm, col_hbm, ew_hbm, zp, acc,
                row_bufs, col_bufs, ew_bufs, gvs, sems, eoff, feat):
    sem_r, sem_c, sem_e, sem_g, sem_s = sems
    njobs = N_CHUNKS

    def idx_start(j):
        b = j % NSLOT
        base = v * E_PAD + c * (E_PAD // 2) + s * PER_TILE + j * CHUNK
        return (
            pltpu.async_copy(row_hbm.at[pl.ds(base, CHUNK)], row_bufs[b], sem_r[b]),
            pltpu.async_copy(col_hbm.at[pl.ds(base, CHUNK)], col_bufs[b], sem_c[b]),
            pltpu.async_copy(ew_hbm.at[pl.ds(base, CHUNK)], ew_bufs[b], sem_e[b]),
        )

    def gather_start(j):
        b = j % NSLOT
        return pltpu.async_copy(zp.at[row_bufs[b]], gvs[b], sem_g[b])

    idx_d = {0: idx_start(0), 1: idx_start(1)}
    for d in idx_d.pop(0):
        d.wait()
    gat_d = {0: gather_start(0)}
    sca_d = {}

    for j in range(njobs):
        b = j % NSLOT
        if j + 1 < njobs:
            if j - 3 in sca_d:
                sca_d.pop(j - 3).wait()
            for d in idx_d.pop(j + 1):
                d.wait()
            gat_d[j + 1] = gather_start(j + 1)
        if j + 2 < njobs:
            if j - 2 in sca_d:
                sca_d.pop(j - 2).wait()
            idx_d[j + 2] = idx_start(j + 2)
        gat_d.pop(j).wait()

        @plsc.parallel_loop(0, CHUNK // 2, unroll=8)
        def _(i):
            w = plsc.load_gather(ew_bufs[b], [2 * i + eoff])
            g = _ld2(gvs[b], i, eoff, feat)
            _st2(gvs[b], i, g * w, eoff, feat)
        sca_d[j] = pltpu.async_copy(gvs[b], acc.at[col_bufs[b]],
                                    sem_s[b], add=True)

    for d in sca_d.values():
        d.wait()


_PROP_SCRATCH = (
    [pltpu.VMEM_SHARED((N, W), jnp.float32)] * 2
    + [pltpu.VMEM((CHUNK,), jnp.int32)] * 4
    + [pltpu.VMEM((CHUNK,), jnp.int32)] * 4
    + [pltpu.VMEM((CHUNK,), jnp.float32)] * 4
    + [pltpu.VMEM((CHUNK, W), jnp.float32)] * 4
    + [pltpu.SemaphoreType.DMA] * 20
    + [pltpu.VMEM((RBUF, W), jnp.float32)] * 3
)


@functools.partial(
    pl.kernel,
    out_type=jax.ShapeDtypeStruct((2, V, N, W), jnp.float32),
    mesh=_sc_mesh(),
    compiler_params=_SC_PARAMS,
    scratch_types=_PROP_SCRATCH,
)
def _prop1_kernel(degp_hbm, xw_hbm, row_hbm, col_hbm, ew_hbm, zero8_hbm,
                  out_hbm, zp, acc, *rest):
    (row_bufs, col_bufs, ew_bufs, gvs) = (rest[0:4], rest[4:8], rest[8:12],
                                          rest[12:16])
    sems = (rest[16:20], rest[20:24], rest[24:28], rest[28:32], rest[32:36])
    p0b, p1b, xwb = rest[36:39]
    c = lax.axis_index("c")
    s = lax.axis_index("s")
    lane = lax.iota(jnp.int32, 16)
    eoff = lane >> 3
    feat = lane & 7

    for v in range(V):
        pltpu.sync_copy(zero8_hbm.at[pl.ds(s * ROWS_PER, ROWS_PER), :],
                        acc.at[pl.ds(s * ROWS_PER, ROWS_PER), :])

        @pl.when(s == 15)
        def _():
            pltpu.sync_copy(zero8_hbm.at[pl.ds(TAIL_START, TAIL), :],
                            acc.at[pl.ds(TAIL_START, TAIL), :])

        for start, nrows, _boff in _ranges(s):
            def do_range(start=start, nrows=nrows, v=v):
                pltpu.sync_copy(degp_hbm.at[0, v, pl.ds(start, nrows), :],
                                p0b.at[pl.ds(0, nrows), :])
                pltpu.sync_copy(degp_hbm.at[1, v, pl.ds(start, nrows), :],
                                p1b.at[pl.ds(0, nrows), :])
                pltpu.sync_copy(xw_hbm.at[v, pl.ds(start, nrows), :],
                                xwb.at[pl.ds(0, nrows), :])

                @plsc.parallel_loop(0, nrows * W // 16, unroll=4)
                def _(j):
                    d = _ld2(p0b, j, eoff, feat) + _ld2(p1b, j, eoff, feat) + 1.0
                    y = _rsqrt16(d)
                    _st2(xwb, j, y * _ld2(xwb, j, eoff, feat), eoff, feat)

                pltpu.sync_copy(xwb.at[pl.ds(0, nrows), :],
                                zp.at[pl.ds(start, nrows), :])

            if _boff == 0:
                do_range()
            else:
                pl.when(s == 15)(do_range)

        plsc.subcore_barrier()
        _edge_sweep(c, s, v, row_hbm, col_hbm, ew_hbm, zp, acc,
                    row_bufs, col_bufs, ew_bufs, gvs, sems, eoff, feat)
        plsc.subcore_barrier()

        pltpu.sync_copy(acc.at[pl.ds(s * ROWS_PER, ROWS_PER), :],
                        out_hbm.at[c, v, pl.ds(s * ROWS_PER, ROWS_PER), :])

        @pl.when(s == 15)
        def _(v=v):
            pltpu.sync_copy(acc.at[pl.ds(TAIL_START, TAIL), :],
                            out_hbm.at[c, v, pl.ds(TAIL_START, TAIL), :])


@functools.partial(
    pl.kernel,
    out_type=jax.ShapeDtypeStruct((2, V, N, W), jnp.float32),
    mesh=_sc_mesh(),
    compiler_params=_SC_PARAMS,
    scratch_types=_PROP_SCRATCH + [
        pltpu.VMEM((RBUF, W), jnp.float32),
        pltpu.VMEM((RBUF, W), jnp.float32),
        pltpu.VMEM((RBUF * W,), jnp.float32),
        pltpu.VMEM((RBUF * W,), jnp.float32),
        pltpu.VMEM((16,), jnp.float32),
    ],
)
def _prop2_kernel(degp_hbm, xw_hbm, t1p_hbm, b1x_hbm, row_hbm, col_hbm,
                  ew_hbm, zero8_hbm, out_hbm, zp, acc, *rest):
    (row_bufs, col_bufs, ew_bufs, gvs) = (rest[0:4], rest[4:8], rest[8:12],
                                          rest[12:16])
    sems = (rest[16:20], rest[20:24], rest[24:28], rest[28:32], rest[32:36])
    p0b, p1b, xwb, t0b, t1b, dis_t, hh, b1b = rest[36:44]
    c = lax.axis_index("c")
    s = lax.axis_index("s")
    lane = lax.iota(jnp.int32, 16)
    eoff = lane >> 3
    feat = lane & 7

    for v in range(V):
        pltpu.sync_copy(zero8_hbm.at[pl.ds(s * ROWS_PER, ROWS_PER), :],
                        acc.at[pl.ds(s * ROWS_PER, ROWS_PER), :])

        @pl.when(s == 15)
        def _():
            pltpu.sync_copy(zero8_hbm.at[pl.ds(TAIL_START, TAIL), :],
                            acc.at[pl.ds(TAIL_START, TAIL), :])

        pltpu.sync_copy(b1x_hbm.at[v], b1b)

        for start, nrows, boff in _ranges(s):
            def do_range(start=start, nrows=nrows, boff=boff, v=v):
                pltpu.sync_copy(degp_hbm.at[0, v, pl.ds(start, nrows), :],
                                p0b.at[pl.ds(0, nrows), :])
                pltpu.sync_copy(degp_hbm.at[1, v, pl.ds(start, nrows), :],
                                p1b.at[pl.ds(0, nrows), :])
                pltpu.sync_copy(xw_hbm.at[v, pl.ds(start, nrows), :],
                                xwb.at[pl.ds(0, nrows), :])
                pltpu.sync_copy(t1p_hbm.at[0, v, pl.ds(start, nrows), :],
                                t0b.at[pl.ds(0, nrows), :])
                pltpu.sync_copy(t1p_hbm.at[1, v, pl.ds(start, nrows), :],
                                t1b.at[pl.ds(0, nrows), :])
                b1v = b1b[...]
                fbase = boff * W

                @plsc.parallel_loop(0, nrows * W // 16, unroll=4)
                def _(j):
                    d = _ld2(p0b, j, eoff, feat) + _ld2(p1b, j, eoff, feat) + 1.0
                    y = _rsqrt16(d)
                    y2 = y * y
                    pre = (y * (_ld2(t0b, j, eoff, feat) + _ld2(t1b, j, eoff, feat))
                           + y2 * _ld2(xwb, j, eoff, feat) + b1v)
                    h = jnp.maximum(pre, 0.0)
                    _st2(xwb, j, y * h, eoff, feat)
                    dis_t[pl.ds(fbase + 16 * j, 16)] = y
                    hh[pl.ds(fbase + 16 * j, 16)] = 0.5 * y2 * h

                pltpu.sync_copy(xwb.at[pl.ds(0, nrows), :],
                                zp.at[pl.ds(start, nrows), :])

            if boff == 0:
                do_range()
            else:
                pl.when(s == 15)(do_range)

        plsc.subcore_barrier()
        _edge_sweep(c, s, v, row_hbm, col_hbm, ew_hbm, zp, acc,
                    row_bufs, col_bufs, ew_bufs, gvs, sems, eoff, feat)
        plsc.subcore_barrier()

        for start, nrows, boff in _ranges(s):
            def do_range(start=start, nrows=nrows, boff=boff, v=v):
                pltpu.sync_copy(acc.at[pl.ds(start, nrows), :],
                                p0b.at[pl.ds(0, nrows), :])
                fbase = boff * W

                @plsc.parallel_loop(0, nrows * W // 16, unroll=4)
                def _(j):
                    y = dis_t[pl.ds(fbase + 16 * j, 16)]
                    z = hh[pl.ds(fbase + 16 * j, 16)]
                    _st2(p0b, j, y * _ld2(p0b, j, eoff, feat) + z, eoff, feat)

                pltpu.sync_copy(p0b.at[pl.ds(0, nrows), :],
                                out_hbm.at[c, v, pl.ds(start, nrows), :])

            if boff == 0:
                do_range()
            else:
                pl.when(s == 15)(do_range)


BN = 2000
E_TAIL = E_PAD - E


def _repack_body(ei_ref, ew_ref, row_ref, col_ref, ewo_ref):
    for v in range(V):
        b = v * E_PAD
        row_ref[pl.ds(b, E)] = ei_ref[v, 0, :]
        row_ref[pl.ds(b + E, E_TAIL)] = jnp.zeros((E_TAIL,), jnp.int32)
        col_ref[pl.ds(b, E)] = ei_ref[v, 1, :]
        col_ref[pl.ds(b + E, E_TAIL)] = jnp.zeros((E_TAIL,), jnp.int32)
        ewo_ref[pl.ds(b, E)] = ew_ref[v, :]
        ewo_ref[pl.ds(b + E, E_TAIL)] = jnp.zeros((E_TAIL,), jnp.float32)


def _tc_repack(ei, ew):
    return pl.pallas_call(
        _repack_body,
        out_shape=[
            jax.ShapeDtypeStruct((V * E_PAD,), jnp.int32),
            jax.ShapeDtypeStruct((V * E_PAD,), jnp.int32),
            jax.ShapeDtypeStruct((V * E_PAD,), jnp.float32),
        ],
    )(ei, ew)


def _xw_body(x_ref, w_ref, o_ref):
    o_ref[...] = jnp.dot(x_ref[...], w_ref[0],
                         preferred_element_type=jnp.float32)[None]


def _tc_xw(x, w1p):
    return pl.pallas_call(
        _xw_body,
        grid=(V, N // BN),
        in_specs=[
            pl.BlockSpec((BN, D_IN), lambda v, n: (n, 0)),
            pl.BlockSpec((1, D_IN, W), lambda v, n: (v, 0, 0)),
        ],
        out_specs=pl.BlockSpec((1, BN, W), lambda v, n: (v, n, 0)),
        out_shape=jax.ShapeDtypeStruct((V, N, W), jnp.float32),
    )(x, w1p)


def _fuse_body(m_ref, w2_ref, b2_ref, o_ref):
    acc = b2_ref[0, 0] + b2_ref[1, 0] + b2_ref[2, 0]
    acc = jnp.broadcast_to(acc[None], (BN, W))
    for v in range(V):
        p2 = m_ref[0, v] + m_ref[1, v]
        acc = acc + jnp.dot(p2, w2_ref[v], preferred_element_type=jnp.float32)
    o_ref[...] = acc


def _tc_fuse(m, w2p, b2p):
    return pl.pallas_call(
        _fuse_body,
        grid=(N // BN,),
        in_specs=[
            pl.BlockSpec((2, V, BN, W), lambda n: (0, 0, n, 0)),
            pl.BlockSpec((V, W, W), lambda n: (0, 0, 0)),
            pl.BlockSpec((V, 1, W), lambda n: (0, 0, 0)),
        ],
        out_specs=pl.BlockSpec((BN, W), lambda n: (n, 0)),
        out_shape=jax.ShapeDtypeStruct((N, W), jnp.float32),
    )(m, w2p, b2p)


def kernel(x, edge_indices, edge_weights, W1, b1, W2, b2, attention):
    del attention
    f32 = jnp.float32

    row, col, ew = _tc_repack(edge_indices, edge_weights)

    zero8 = jnp.zeros((N, W), f32)
    w1p = jnp.pad(W1, ((0, 0), (0, 0), (0, W - H)))
    b1x = jnp.tile(jnp.pad(b1, ((0, 0), (0, W - H))), (1, 2))
    w2p = jnp.pad(W2, ((0, 0), (0, W - H), (0, W - D_OUT)))
    b2p = jnp.pad(b2, ((0, 0), (0, W - D_OUT)))[:, None, :]

    xw = _tc_xw(x, w1p)
    degp = _deg_kernel(col, ew, zero8)
    t1p = _prop1_kernel(degp, xw, row, col, ew, zero8)
    m = _prop2_kernel(degp, xw, t1p, b1x, row, col, ew, zero8)
    fused = _tc_fuse(m, w2p, b2p)
    return fused[:, :D_OUT]
